# baseline (speedup 1.0000x reference)
import jax
import jax.numpy as jnp
from jax import lax
from jax.experimental import pallas as pl
from jax.experimental.pallas import tpu as pltpu
from jax.experimental.pallas import tpu_sc as plsc

N = 10000
D = 128
E = 320000
NC = 2
NS = 16
NW = NC * NS
EPW = E // NW
NR = 78
TAIL = EPW - NR * 128
CH = 64
QR = 4
NP = 10240
RPT = NP // NS


def _spmm_partials(feat, src_m, src_t, dst_m, dst_t, zrows):
    mesh = plsc.VectorSubcoreMesh(core_axis_name="c", subcore_axis_name="s")

    @pl.kernel(
        out_type=jax.ShapeDtypeStruct((NC, NP, D), jnp.float32),
        mesh=mesh,
        scratch_types=[
            pltpu.VMEM((QR, 128), jnp.int32),
            pltpu.VMEM((QR, 128), jnp.int32),
            pltpu.VMEM((TAIL,), jnp.int32),
            pltpu.VMEM((TAIL,), jnp.int32),
            pltpu.VMEM((CH, D), jnp.float32),
            pltpu.VMEM((CH, D), jnp.float32),
            pltpu.VMEM((CH, D), jnp.float32),
            pltpu.VMEM((CH, D), jnp.float32),
            pltpu.VMEM((CH, D), jnp.float32),
            pltpu.VMEM((TAIL, D), jnp.float32),
            pltpu.VMEM_SHARED((NP, D), jnp.float32),
            pltpu.SemaphoreType.DMA,
            pltpu.SemaphoreType.DMA,
            pltpu.SemaphoreType.DMA,
            pltpu.SemaphoreType.DMA,
            pltpu.SemaphoreType.DMA,
            pltpu.SemaphoreType.DMA,
            pltpu.SemaphoreType.DMA,
            pltpu.SemaphoreType.DMA,
            pltpu.SemaphoreType.DMA,
            pltpu.SemaphoreType.DMA,
            pltpu.SemaphoreType.DMA,
            pltpu.SemaphoreType.DMA,
            pltpu.SemaphoreType.DMA,
        ],
    )
    def k(feat_hbm, srcm_hbm, srct_hbm, dstm_hbm, dstt_hbm, zrows_hbm,
          out_hbm, s_ring, d_ring, stail, dtail, rb0, rb1, rb2, rb3, rb4,
          rtail, accum, semz, semfs, semfd,
          g0, g1, g2, g3, g4, s0, s1, s2, s3, s4):
        c = lax.axis_index("c")
        s = lax.axis_index("s")
        wid = c * NS + s

        zcp = pltpu.make_async_copy(
            zrows_hbm.at[pl.ds(s * RPT, RPT)],
            accum.at[pl.ds(s * RPT, RPT)], semz)
        zcp.start()

        pltpu.sync_copy(srcm_hbm.at[wid, pl.ds(0, QR - 1)],
                        s_ring.at[pl.ds(0, QR - 1)])
        pltpu.sync_copy(dstm_hbm.at[wid, pl.ds(0, QR - 1)],
                        d_ring.at[pl.ds(0, QR - 1)])
        pltpu.sync_copy(srct_hbm.at[wid], stail)
        pltpu.sync_copy(dstt_hbm.at[wid], dtail)

        def gsrc(t):
            return feat_hbm.at[
                s_ring.at[(t // 2) % QR, pl.ds((t % 2) * CH, CH)]]

        def dsl(t):
            return accum.at[
                d_ring.at[(t // 2) % QR, pl.ds((t % 2) * CH, CH)]]

        bufs = (rb0, rb1, rb2, rb3, rb4)
        gsems = (g0, g1, g2, g3, g4)
        ssems = (s0, s1, s2, s3, s4)
        NCH = 2 * NR

        for u in range(4):
            pltpu.async_copy(gsrc(u), bufs[u], gsems[u])

        zcp.wait()
        plsc.subcore_barrier()

        def fetch_desc(r):
            rf = jnp.minimum(r, NR - 1)
            a = pltpu.make_async_copy(
                srcm_hbm.at[wid, rf], s_ring.at[r % QR], semfs)
            b = pltpu.make_async_copy(
                dstm_hbm.at[wid, rf], d_ring.at[r % QR], semfd)
            return a, b

        def step(t, b):
            nb = (b + 4) % 5
            r = t // 2
            pltpu.make_async_copy(gsrc(t), bufs[b], gsems[b]).wait()
            pltpu.async_copy(bufs[b], dsl(t), ssems[b], add=True)

            @pl.when(t >= 1)
            def _():
                pltpu.make_async_copy(bufs[nb], dsl(t - 1), ssems[nb]).wait()

            @pl.when(t % 2 == 0)
            def _():
                @pl.when(t >= 2)
                def _():
                    a, bb = fetch_desc(r + 2)
                    a.wait()
                    bb.wait()
                a, bb = fetch_desc(r + 3)
                a.start()
                bb.start()

            t4 = jnp.minimum(t + 4, NCH - 1)
            pltpu.async_copy(gsrc(t4), bufs[nb], gsems[nb])

        @pl.loop(0, (NCH - 1) // 5)
        def _(i):
            for b in range(5):
                step(5 * i + b, b)

        tl = NCH - 1
        pltpu.make_async_copy(gsrc(tl), bufs[0], gsems[0]).wait()
        pltpu.async_copy(bufs[0], dsl(tl), ssems[0], add=True)
        pltpu.make_async_copy(bufs[4], dsl(tl - 1), ssems[4]).wait()

        pltpu.make_async_copy(bufs[0], dsl(tl), ssems[0]).wait()
        for b in range(1, 4):
            pltpu.make_async_copy(gsrc(tl), bufs[b], gsems[b]).wait()
        a, bb = fetch_desc(NR - 1)
        a.wait()
        bb.wait()

        pltpu.sync_copy(feat_hbm.at[stail], rtail)
        pltpu.sync_copy(rtail, accum.at[dtail], add=True)

        plsc.subcore_barrier()

        pltpu.sync_copy(
            accum.at[pl.ds(s * RPT, RPT)],
            out_hbm.at[c].at[pl.ds(s * RPT, RPT)],
        )

    return k(feat, src_m, src_t, dst_m, dst_t, zrows)


def _layer_tc(xin, pq, wa, wb, do_relu):
    bm = 1000

    def body(x_ref, p0_ref, p1_ref, wa_ref, wb_ref, o_ref):
        acc = jnp.dot(x_ref[...], wa_ref[...],
                      preferred_element_type=jnp.float32)
        acc = acc + jnp.dot(p0_ref[0] + p1_ref[0], wb_ref[...],
                            preferred_element_type=jnp.float32)
        if do_relu:
            acc = jnp.maximum(acc, 0.0)
        o_ref[...] = acc

    return pl.pallas_call(
        body,
        grid=(N // bm,),
        in_specs=[
            pl.BlockSpec((bm, D), lambda i: (i, 0)),
            pl.BlockSpec((1, bm, D), lambda i: (0, i, 0)),
            pl.BlockSpec((1, bm, D), lambda i: (1, i, 0)),
            pl.BlockSpec((D, D), lambda i: (0, 0)),
            pl.BlockSpec((D, D), lambda i: (0, 0)),
        ],
        out_specs=pl.BlockSpec((bm, D), lambda i: (i, 0)),
        out_shape=jax.ShapeDtypeStruct((N, D), jnp.float32),
    )(xin, pq, pq, wa, wb)


def kernel(X, edge_index, W1, W2):
    e32 = edge_index.astype(jnp.int32).reshape(2, NW, EPW)
    src_m = e32[0, :, : NR * 128].reshape(NW, NR, 128)
    src_t = e32[0, :, NR * 128:]
    dst_m = e32[1, :, : NR * 128].reshape(NW, NR, 128)
    dst_t = e32[1, :, NR * 128:]
    w1t = W1.T
    w2t = W2.T

    zrows = jnp.zeros((NP, D), jnp.float32)

    p = _spmm_partials(X, src_m, src_t, dst_m, dst_t, zrows)
    h = _layer_tc(X, p, w1t[:D], w1t[D:], True)
    q = _spmm_partials(h, src_m, src_t, dst_m, dst_t, zrows)
    out = _layer_tc(h, q, w2t[:D], w2t[D:], False)
    return out


# speedup vs baseline: 1.0168x; 1.0168x over previous
import jax
import jax.numpy as jnp
from jax import lax
from jax.experimental import pallas as pl
from jax.experimental.pallas import tpu as pltpu
from jax.experimental.pallas import tpu_sc as plsc

N = 10000
D = 128
E = 320000
NC = 2
NS = 16
NW = NC * NS
EPW = E // NW
NR = 78
TAIL = EPW - NR * 128
CH = 64
QR = 4
NP = 10240
RPT = NP // NS


def _spmm_partials(feat, src_m, src_t, dst_m, dst_t):
    mesh = plsc.VectorSubcoreMesh(core_axis_name="c", subcore_axis_name="s")

    @pl.kernel(
        out_type=jax.ShapeDtypeStruct((NC, NP, D), jnp.float32),
        mesh=mesh,
        scratch_types=[
            pltpu.VMEM((QR, 128), jnp.int32),
            pltpu.VMEM((QR, 128), jnp.int32),
            pltpu.VMEM((TAIL,), jnp.int32),
            pltpu.VMEM((TAIL,), jnp.int32),
            pltpu.VMEM((CH, D), jnp.float32),
            pltpu.VMEM((CH, D), jnp.float32),
            pltpu.VMEM((CH, D), jnp.float32),
            pltpu.VMEM((CH, D), jnp.float32),
            pltpu.VMEM((CH, D), jnp.float32),
            pltpu.VMEM((TAIL, D), jnp.float32),
            pltpu.VMEM_SHARED((NP, D), jnp.float32),
            pltpu.SemaphoreType.DMA,
            pltpu.SemaphoreType.DMA,
            pltpu.SemaphoreType.DMA,
            pltpu.SemaphoreType.DMA,
            pltpu.SemaphoreType.DMA,
            pltpu.SemaphoreType.DMA,
            pltpu.SemaphoreType.DMA,
            pltpu.SemaphoreType.DMA,
            pltpu.SemaphoreType.DMA,
            pltpu.SemaphoreType.DMA,
            pltpu.SemaphoreType.DMA,
            pltpu.SemaphoreType.DMA,
        ],
    )
    def k(feat_hbm, srcm_hbm, srct_hbm, dstm_hbm, dstt_hbm, out_hbm,
          s_ring, d_ring, stail, dtail, rb0, rb1, rb2, rb3, rb4,
          rtail, accum, semfs, semfd,
          g0, g1, g2, g3, g4, s0, s1, s2, s3, s4):
        c = lax.axis_index("c")
        s = lax.axis_index("s")
        wid = c * NS + s

        pltpu.sync_copy(srcm_hbm.at[wid, pl.ds(0, QR - 1)],
                        s_ring.at[pl.ds(0, QR - 1)])
        pltpu.sync_copy(dstm_hbm.at[wid, pl.ds(0, QR - 1)],
                        d_ring.at[pl.ds(0, QR - 1)])
        pltpu.sync_copy(srct_hbm.at[wid], stail)
        pltpu.sync_copy(dstt_hbm.at[wid], dtail)

        def gsrc(t):
            return feat_hbm.at[
                s_ring.at[(t // 2) % QR, pl.ds((t % 2) * CH, CH)]]

        def dsl(t):
            return accum.at[
                d_ring.at[(t // 2) % QR, pl.ds((t % 2) * CH, CH)]]

        bufs = (rb0, rb1, rb2, rb3, rb4)
        gsems = (g0, g1, g2, g3, g4)
        ssems = (s0, s1, s2, s3, s4)
        NCH = 2 * NR

        for u in range(4):
            pltpu.async_copy(gsrc(u), bufs[u], gsems[u])

        zeros16 = jnp.zeros((16,), jnp.float32)

        @pl.loop(0, CH)
        def _(r):
            for j in range(D // 16):
                rb4[r, pl.ds(j * 16, 16)] = zeros16

        @pl.loop(0, RPT // CH)
        def _(t):
            pltpu.sync_copy(rb4, accum.at[pl.ds(s * RPT + t * CH, CH)])

        plsc.subcore_barrier()

        def fetch_desc(r):
            rf = jnp.minimum(r, NR - 1)
            a = pltpu.make_async_copy(
                srcm_hbm.at[wid, rf], s_ring.at[r % QR], semfs)
            b = pltpu.make_async_copy(
                dstm_hbm.at[wid, rf], d_ring.at[r % QR], semfd)
            return a, b

        def step(t, b):
            nb = (b + 4) % 5
            r = t // 2
            pltpu.make_async_copy(gsrc(t), bufs[b], gsems[b]).wait()
            pltpu.async_copy(bufs[b], dsl(t), ssems[b], add=True)

            @pl.when(t >= 1)
            def _():
                pltpu.make_async_copy(bufs[nb], dsl(t - 1), ssems[nb]).wait()

            @pl.when(t % 2 == 0)
            def _():
                @pl.when(t >= 2)
                def _():
                    a, bb = fetch_desc(r + 2)
                    a.wait()
                    bb.wait()
                a, bb = fetch_desc(r + 3)
                a.start()
                bb.start()

            t4 = jnp.minimum(t + 4, NCH - 1)
            pltpu.async_copy(gsrc(t4), bufs[nb], gsems[nb])

        @pl.loop(0, (NCH - 1) // 5)
        def _(i):
            for b in range(5):
                step(5 * i + b, b)

        tl = NCH - 1
        pltpu.make_async_copy(gsrc(tl), bufs[0], gsems[0]).wait()
        pltpu.async_copy(bufs[0], dsl(tl), ssems[0], add=True)
        pltpu.make_async_copy(bufs[4], dsl(tl - 1), ssems[4]).wait()

        pltpu.make_async_copy(bufs[0], dsl(tl), ssems[0]).wait()
        for b in range(1, 4):
            pltpu.make_async_copy(gsrc(tl), bufs[b], gsems[b]).wait()
        a, bb = fetch_desc(NR - 1)
        a.wait()
        bb.wait()

        pltpu.sync_copy(feat_hbm.at[stail], rtail)
        pltpu.sync_copy(rtail, accum.at[dtail], add=True)

        plsc.subcore_barrier()

        pltpu.sync_copy(
            accum.at[pl.ds(s * RPT, RPT)],
            out_hbm.at[c].at[pl.ds(s * RPT, RPT)],
        )

    return k(feat, src_m, src_t, dst_m, dst_t)


def _mm_tc(xin, wa):
    bm = 1000

    def body(x_ref, wa_ref, o_ref):
        o_ref[...] = jnp.dot(x_ref[...], wa_ref[...],
                             preferred_element_type=jnp.float32)

    return pl.pallas_call(
        body,
        grid=(N // bm,),
        in_specs=[
            pl.BlockSpec((bm, D), lambda i: (i, 0)),
            pl.BlockSpec((D, D), lambda i: (0, 0)),
        ],
        out_specs=pl.BlockSpec((bm, D), lambda i: (i, 0)),
        out_shape=jax.ShapeDtypeStruct((N, D), jnp.float32),
    )(xin, wa)


def _layer_tc(xa, pq, wb, do_relu):
    bm = 1000

    def body(xa_ref, p0_ref, p1_ref, wb_ref, o_ref):
        acc = xa_ref[...] + jnp.dot(p0_ref[0] + p1_ref[0], wb_ref[...],
                                    preferred_element_type=jnp.float32)
        if do_relu:
            acc = jnp.maximum(acc, 0.0)
        o_ref[...] = acc

    return pl.pallas_call(
        body,
        grid=(N // bm,),
        in_specs=[
            pl.BlockSpec((bm, D), lambda i: (i, 0)),
            pl.BlockSpec((1, bm, D), lambda i: (0, i, 0)),
            pl.BlockSpec((1, bm, D), lambda i: (1, i, 0)),
            pl.BlockSpec((D, D), lambda i: (0, 0)),
        ],
        out_specs=pl.BlockSpec((bm, D), lambda i: (i, 0)),
        out_shape=jax.ShapeDtypeStruct((N, D), jnp.float32),
    )(xa, pq, pq, wb)


def kernel(X, edge_index, W1, W2):
    e32 = edge_index.astype(jnp.int32).reshape(2, NW, EPW)
    src_m = e32[0, :, : NR * 128].reshape(NW, NR, 128)
    src_t = e32[0, :, NR * 128:]
    dst_m = e32[1, :, : NR * 128].reshape(NW, NR, 128)
    dst_t = e32[1, :, NR * 128:]
    w1t = W1.T
    w2t = W2.T

    p = _spmm_partials(X, src_m, src_t, dst_m, dst_t)
    xa = _mm_tc(X, w1t[:D])
    h = _layer_tc(xa, p, w1t[D:], True)
    q = _spmm_partials(h, src_m, src_t, dst_m, dst_t)
    ha = _mm_tc(h, w2t[:D])
    out = _layer_tc(ha, q, w2t[D:], False)
    return out
